# baseline (speedup 1.0000x reference)
import functools

import jax
import jax.numpy as jnp
from jax import lax
from jax.experimental import pallas as pl
from jax.experimental.pallas import tpu as pltpu
from jax.experimental.pallas import tpu_sc as plsc

_B = 4096
_F = 16
_N = 1000000
_RPG = 128 // _F

_info = plsc.get_sparse_core_info()
_NC = _info.num_cores
_NS = _info.num_subcores
_L = _info.num_lanes
_NW = _NC * _NS
_BPW = _B // _NW

_mesh = plsc.VectorSubcoreMesh(core_axis_name="c", subcore_axis_name="s")


@functools.partial(
    pl.kernel,
    mesh=_mesh,
    out_type=[
        jax.ShapeDtypeStruct((_B,), jnp.float32),
        jax.ShapeDtypeStruct((_B,), jnp.float32),
    ],
    scratch_types=[
        pltpu.VMEM((_BPW,), jnp.int32),
        pltpu.VMEM((_BPW,), jnp.int32),
        pltpu.VMEM((_BPW,), jnp.int32),
        pltpu.VMEM((_BPW,), jnp.int32),
        pltpu.VMEM((_BPW, 128), jnp.float32),
        pltpu.VMEM((_BPW, 128), jnp.float32),
        pltpu.VMEM((_BPW,), jnp.float32),
        pltpu.VMEM((_BPW,), jnp.float32),
        pltpu.VMEM((_BPW,), jnp.float32),
        pltpu.VMEM((_BPW,), jnp.float32),
        pltpu.SemaphoreType.DMA,
    ],
    compiler_params=pltpu.CompilerParams(needs_layout_passes=False),
)
def _sc_gather_dot(x1_hbm, x2_hbm, emb_hbm, bias_hbm, s_hbm, b_hbm,
                   idx1_v, idx2_v, gid1_v, gid2_v, rows1_v, rows2_v,
                   b1_v, b2_v, s_v, b_v, sem):
    wid = lax.axis_index("s") * _NC + lax.axis_index("c")
    base = wid * _BPW
    pltpu.sync_copy(x1_hbm.at[pl.ds(base, _BPW)], idx1_v)
    pltpu.sync_copy(x2_hbm.at[pl.ds(base, _BPW)], idx2_v)
    for g in range(_BPW // _L):
        sl = pl.ds(g * _L, _L)
        gid1_v[sl] = idx1_v[sl] >> 3
        gid2_v[sl] = idx2_v[sl] >> 3
    c1 = pltpu.async_copy(emb_hbm.at[gid1_v], rows1_v, sem)
    c2 = pltpu.async_copy(emb_hbm.at[gid2_v], rows2_v, sem)
    c3 = pltpu.async_copy(bias_hbm.at[idx1_v], b1_v, sem)
    c4 = pltpu.async_copy(bias_hbm.at[idx2_v], b2_v, sem)
    c1.wait()
    c2.wait()
    c3.wait()
    c4.wait()
    lane = lax.iota(jnp.int32, _L)
    for g in range(_BPW // _L):
        sl = pl.ds(g * _L, _L)
        o1 = (idx1_v[sl] & 7) << 4
        o2 = (idx2_v[sl] & 7) << 4
        svec = jnp.zeros((_L,), jnp.float32)
        for l in range(_L):
            j = g * _L + l
            e1 = rows1_v[j, pl.ds(pl.multiple_of(o1[l], 16), _F)]
            e2 = rows2_v[j, pl.ds(pl.multiple_of(o2[l], 16), _F)]
            svec = jnp.where(lane == l, jnp.sum(e1 * e2), svec)
        s_v[sl] = svec
        b_v[sl] = b1_v[sl] + b2_v[sl]
    pltpu.sync_copy(s_v, s_hbm.at[pl.ds(base, _BPW)])
    pltpu.sync_copy(b_v, b_hbm.at[pl.ds(base, _BPW)])


_RB = 256


def _bcast_body(b_ref, s_ref, o_ref):
    o_ref[...] = b_ref[...] + s_ref[...]


def _broadcast_add(b_col, s_row):
    return pl.pallas_call(
        _bcast_body,
        grid=(_B // _RB,),
        in_specs=[
            pl.BlockSpec((_RB, 1), lambda i: (i, 0)),
            pl.BlockSpec((1, _B), lambda i: (0, 0)),
        ],
        out_specs=pl.BlockSpec((_RB, _B), lambda i: (i, 0)),
        out_shape=jax.ShapeDtypeStruct((_B, _B), jnp.float32),
    )(b_col, s_row)


def kernel(x1, x2, emb_table, bias_table):
    emb_grouped = emb_table.reshape(_N // _RPG, 128)
    s, b = _sc_gather_dot(x1.astype(jnp.int32), x2.astype(jnp.int32),
                          emb_grouped, bias_table.reshape(-1))
    return _broadcast_add(b.reshape(_B, 1), s.reshape(1, _B))


# speedup vs baseline: 1.4036x; 1.4036x over previous
import functools

import jax
import jax.numpy as jnp
from jax import lax
from jax.experimental import pallas as pl
from jax.experimental.pallas import tpu as pltpu
from jax.experimental.pallas import tpu_sc as plsc

_B = 4096
_F = 16
_NSEM = 8

_info = plsc.get_sparse_core_info()
_NC = _info.num_cores
_NS = _info.num_subcores
_L = _info.num_lanes
_NW = _NC * _NS
_BPW = _B // _NW

_mesh = plsc.VectorSubcoreMesh(core_axis_name="c", subcore_axis_name="s")


@functools.partial(
    pl.kernel,
    mesh=_mesh,
    out_type=[
        jax.ShapeDtypeStruct((_B,), jnp.float32),
        jax.ShapeDtypeStruct((_B,), jnp.float32),
    ],
    scratch_types=[
        pltpu.VMEM((_BPW,), jnp.int32),
        pltpu.VMEM((_BPW,), jnp.int32),
        pltpu.VMEM((_BPW, _F), jnp.float32),
        pltpu.VMEM((_BPW, _F), jnp.float32),
        pltpu.VMEM((_BPW,), jnp.float32),
        pltpu.VMEM((_BPW,), jnp.float32),
        pltpu.VMEM((_BPW,), jnp.float32),
        pltpu.VMEM((_BPW,), jnp.float32),
        [pltpu.SemaphoreType.DMA] * _NSEM,
    ],
    compiler_params=pltpu.CompilerParams(needs_layout_passes=False),
)
def _sc_gather_dot(x1_hbm, x2_hbm, emb_hbm, bias_hbm, s_hbm, b_hbm,
                   idx1_v, idx2_v, rows1_v, rows2_v, b1_v, b2_v, s_v, b_v,
                   sems):
    wid = lax.axis_index("s") * _NC + lax.axis_index("c")
    base = wid * _BPW
    pltpu.sync_copy(x1_hbm.at[pl.ds(base, _BPW)], idx1_v)
    pltpu.sync_copy(x2_hbm.at[pl.ds(base, _BPW)], idx2_v)
    copies = [
        pltpu.async_copy(bias_hbm.at[idx1_v], b1_v, sems[0]),
        pltpu.async_copy(bias_hbm.at[idx2_v], b2_v, sems[1]),
    ]
    n = 2
    for g in range(_BPW // _L):
        iv1 = idx1_v[pl.ds(g * _L, _L)]
        iv2 = idx2_v[pl.ds(g * _L, _L)]
        for l in range(_L):
            j = g * _L + l
            copies.append(pltpu.async_copy(emb_hbm.at[iv1[l]],
                                           rows1_v.at[j], sems[n % _NSEM]))
            n += 1
            copies.append(pltpu.async_copy(emb_hbm.at[iv2[l]],
                                           rows2_v.at[j], sems[n % _NSEM]))
            n += 1
    for c in copies:
        c.wait()
    lane = lax.iota(jnp.int32, _L)
    for g in range(_BPW // _L):
        sl = pl.ds(g * _L, _L)
        svec = jnp.zeros((_L,), jnp.float32)
        for l in range(_L):
            j = g * _L + l
            prod = rows1_v[j, :] * rows2_v[j, :]
            svec = jnp.where(lane == l, jnp.sum(prod), svec)
        s_v[sl] = svec
        b_v[sl] = b1_v[sl] + b2_v[sl]
    pltpu.sync_copy(s_v, s_hbm.at[pl.ds(base, _BPW)])
    pltpu.sync_copy(b_v, b_hbm.at[pl.ds(base, _BPW)])


_RB = 256


def _bcast_body(b_ref, s_ref, o_ref):
    o_ref[...] = b_ref[...] + s_ref[...]


def _broadcast_add(b_col, s_row):
    return pl.pallas_call(
        _bcast_body,
        grid=(_B // _RB,),
        in_specs=[
            pl.BlockSpec((_RB, 1), lambda i: (i, 0)),
            pl.BlockSpec((1, _B), lambda i: (0, 0)),
        ],
        out_specs=pl.BlockSpec((_RB, _B), lambda i: (i, 0)),
        out_shape=jax.ShapeDtypeStruct((_B, _B), jnp.float32),
    )(b_col, s_row)


def kernel(x1, x2, emb_table, bias_table):
    s, b = _sc_gather_dot(x1.astype(jnp.int32), x2.astype(jnp.int32),
                          emb_table, bias_table.reshape(-1))
    return _broadcast_add(b.reshape(_B, 1), s.reshape(1, _B))


# speedup vs baseline: 1.4177x; 1.0100x over previous
import functools

import jax
import jax.numpy as jnp
from jax import lax
from jax.experimental import pallas as pl
from jax.experimental.pallas import tpu as pltpu
from jax.experimental.pallas import tpu_sc as plsc

_B = 4096
_F = 16
_NSEM = 8

_info = plsc.get_sparse_core_info()
_NC = _info.num_cores
_NS = _info.num_subcores
_L = _info.num_lanes
_NW = _NC * _NS
_BPW = _B // _NW

_mesh = plsc.VectorSubcoreMesh(core_axis_name="c", subcore_axis_name="s")


@functools.partial(
    pl.kernel,
    mesh=_mesh,
    out_type=[
        jax.ShapeDtypeStruct((_B,), jnp.float32),
        jax.ShapeDtypeStruct((_B,), jnp.float32),
    ],
    scratch_types=[
        pltpu.VMEM((_BPW,), jnp.int32),
        pltpu.VMEM((_BPW,), jnp.int32),
        pltpu.VMEM((_BPW, _F), jnp.float32),
        pltpu.VMEM((_BPW, _F), jnp.float32),
        pltpu.VMEM((_BPW,), jnp.float32),
        pltpu.VMEM((_BPW,), jnp.float32),
        pltpu.VMEM((_BPW,), jnp.float32),
        pltpu.VMEM((_BPW,), jnp.float32),
        [pltpu.SemaphoreType.DMA] * _NSEM,
    ],
    compiler_params=pltpu.CompilerParams(needs_layout_passes=False),
)
def _sc_gather_dot(x1_hbm, x2_hbm, emb_hbm, bias_hbm, s_hbm, b_hbm,
                   idx1_v, idx2_v, rows1_v, rows2_v, b1_v, b2_v, s_v, b_v,
                   sems):
    wid = lax.axis_index("s") * _NC + lax.axis_index("c")
    base = wid * _BPW
    pltpu.sync_copy(x1_hbm.at[pl.ds(base, _BPW)], idx1_v)
    pltpu.sync_copy(x2_hbm.at[pl.ds(base, _BPW)], idx2_v)
    bias_copies = [
        pltpu.async_copy(bias_hbm.at[idx1_v], b1_v, sems[1]),
        pltpu.async_copy(bias_hbm.at[idx2_v], b2_v, sems[1]),
    ]
    sem = sems[0]

    def issue_body(g, carry):
        g16 = pl.multiple_of(g * _L, _L)
        iv1 = idx1_v[pl.ds(g16, _L)]
        iv2 = idx2_v[pl.ds(g16, _L)]
        for l in range(_L):
            pltpu.async_copy(emb_hbm.at[iv1[l]], rows1_v.at[g16 + l], sem)
            pltpu.async_copy(emb_hbm.at[iv2[l]], rows2_v.at[g16 + l], sem)
        return carry

    lax.fori_loop(0, _BPW // _L, issue_body, 0)

    def drain_body(g, carry):
        g16 = pl.multiple_of(g * _L, _L)
        for l in range(_L):
            pltpu.make_async_copy(emb_hbm.at[0], rows1_v.at[g16 + l],
                                  sem).wait()
            pltpu.make_async_copy(emb_hbm.at[0], rows2_v.at[g16 + l],
                                  sem).wait()
        return carry

    lax.fori_loop(0, _BPW // _L, drain_body, 0)
    for c in bias_copies:
        c.wait()
    lane = lax.iota(jnp.int32, _L)

    def compute_body(g, carry):
        g16 = pl.multiple_of(g * _L, _L)
        sl = pl.ds(g16, _L)
        svec = jnp.zeros((_L,), jnp.float32)
        for l in range(_L):
            prod = rows1_v[g16 + l, :] * rows2_v[g16 + l, :]
            svec = jnp.where(lane == l, jnp.sum(prod), svec)
        s_v[sl] = svec
        b_v[sl] = b1_v[sl] + b2_v[sl]
        return carry

    lax.fori_loop(0, _BPW // _L, compute_body, 0)
    pltpu.sync_copy(s_v, s_hbm.at[pl.ds(base, _BPW)])
    pltpu.sync_copy(b_v, b_hbm.at[pl.ds(base, _BPW)])


_RB = 256


def _bcast_body(b_ref, s_ref, o_ref):
    o_ref[...] = b_ref[...] + s_ref[...]


def _broadcast_add(b_col, s_row):
    return pl.pallas_call(
        _bcast_body,
        grid=(_B // _RB,),
        in_specs=[
            pl.BlockSpec((_RB, 1), lambda i: (i, 0)),
            pl.BlockSpec((1, _B), lambda i: (0, 0)),
        ],
        out_specs=pl.BlockSpec((_RB, _B), lambda i: (i, 0)),
        out_shape=jax.ShapeDtypeStruct((_B, _B), jnp.float32),
    )(b_col, s_row)


def kernel(x1, x2, emb_table, bias_table):
    s, b = _sc_gather_dot(x1.astype(jnp.int32), x2.astype(jnp.int32),
                          emb_table, bias_table.reshape(-1))
    return _broadcast_add(b.reshape(_B, 1), s.reshape(1, _B))


# speedup vs baseline: 4.0810x; 2.8787x over previous
import functools

import jax
import jax.numpy as jnp
from jax import lax
from jax.experimental import pallas as pl
from jax.experimental.pallas import tpu as pltpu
from jax.experimental.pallas import tpu_sc as plsc

_B = 4096
_F = 16

_info = plsc.get_sparse_core_info()
_NC = _info.num_cores
_NS = _info.num_subcores
_L = _info.num_lanes
_NW = _NC * _NS
_BPW = _B // _NW

_mesh = plsc.VectorSubcoreMesh(core_axis_name="c", subcore_axis_name="s")


@functools.partial(
    pl.kernel,
    mesh=_mesh,
    out_type=[
        jax.ShapeDtypeStruct((_B,), jnp.float32),
        jax.ShapeDtypeStruct((_B,), jnp.float32),
    ],
    scratch_types=[
        pltpu.VMEM((_BPW,), jnp.int32),
        pltpu.VMEM((_BPW,), jnp.int32),
        pltpu.VMEM((_F, 128 * _L), jnp.float32),
        pltpu.VMEM((_F, 128 * _L), jnp.float32),
        pltpu.VMEM((_BPW,), jnp.float32),
        pltpu.VMEM((_BPW,), jnp.float32),
        pltpu.VMEM((_BPW,), jnp.float32),
        pltpu.VMEM((_BPW,), jnp.float32),
        pltpu.SemaphoreType.DMA,
        pltpu.SemaphoreType.DMA,
    ],
    compiler_params=pltpu.CompilerParams(needs_layout_passes=False),
)
def _sc_gather_dot(x1_hbm, x2_hbm, embt_hbm, bias_hbm, s_hbm, b_hbm,
                   idx1_v, idx2_v, e1_v, e2_v, b1_v, b2_v, s_v, b_v,
                   sem, bsem):
    wid = lax.axis_index("s") * _NC + lax.axis_index("c")
    base = wid * _BPW
    pltpu.sync_copy(x1_hbm.at[pl.ds(base, _BPW)], idx1_v)
    pltpu.sync_copy(x2_hbm.at[pl.ds(base, _BPW)], idx2_v)
    bias_copies = [
        pltpu.async_copy(bias_hbm.at[idx1_v], b1_v, bsem),
        pltpu.async_copy(bias_hbm.at[idx2_v], b2_v, bsem),
    ]
    lane = lax.iota(jnp.int32, _L)

    def chunk_body(g, carry):
        g16 = pl.multiple_of(g * _L, _L)
        sl = pl.ds(g16, _L)
        iv1 = idx1_v[sl]
        iv2 = idx2_v[sl]
        a1 = (iv1 >> 7) << 7
        a2 = (iv2 >> 7) << 7
        for l in range(_L):
            d = pl.multiple_of(128 * l, 128)
            pltpu.async_copy(
                embt_hbm.at[pl.ds(0, _F),
                            pl.ds(pl.multiple_of(a1[l], 128), 128)],
                e1_v.at[pl.ds(0, _F), pl.ds(d, 128)], sem)
            pltpu.async_copy(
                embt_hbm.at[pl.ds(0, _F),
                            pl.ds(pl.multiple_of(a2[l], 128), 128)],
                e2_v.at[pl.ds(0, _F), pl.ds(d, 128)], sem)
        for l in range(_L):
            d = pl.multiple_of(128 * l, 128)
            pltpu.make_async_copy(
                embt_hbm.at[pl.ds(0, _F), pl.ds(0, 128)],
                e1_v.at[pl.ds(0, _F), pl.ds(d, 128)], sem).wait()
            pltpu.make_async_copy(
                embt_hbm.at[pl.ds(0, _F), pl.ds(0, 128)],
                e2_v.at[pl.ds(0, _F), pl.ds(d, 128)], sem).wait()
        p1 = 128 * lane + (iv1 & 127)
        p2 = 128 * lane + (iv2 & 127)
        acc = jnp.zeros((_L,), jnp.float32)
        for k in range(_F):
            kvec = jnp.full((_L,), k, jnp.int32)
            e1 = plsc.load_gather(e1_v, [kvec, p1])
            e2 = plsc.load_gather(e2_v, [kvec, p2])
            acc = acc + e1 * e2
        s_v[sl] = acc
        b_v[sl] = b1_v[sl] + b2_v[sl]
        return carry

    lax.fori_loop(0, _BPW // _L, chunk_body, 0)
    for c in bias_copies:
        c.wait()
    pltpu.sync_copy(s_v, s_hbm.at[pl.ds(base, _BPW)])
    pltpu.sync_copy(b_v, b_hbm.at[pl.ds(base, _BPW)])


_RB = 256


def _bcast_body(b_ref, s_ref, o_ref):
    o_ref[...] = b_ref[...] + s_ref[...]


def _broadcast_add(b_col, s_row):
    return pl.pallas_call(
        _bcast_body,
        grid=(_B // _RB,),
        in_specs=[
            pl.BlockSpec((_RB, 1), lambda i: (i, 0)),
            pl.BlockSpec((1, _B), lambda i: (0, 0)),
        ],
        out_specs=pl.BlockSpec((_RB, _B), lambda i: (i, 0)),
        out_shape=jax.ShapeDtypeStruct((_B, _B), jnp.float32),
    )(b_col, s_row)


def kernel(x1, x2, emb_table, bias_table):
    s, b = _sc_gather_dot(x1.astype(jnp.int32), x2.astype(jnp.int32),
                          emb_table.T, bias_table.reshape(-1))
    return _broadcast_add(b.reshape(_B, 1), s.reshape(1, _B))
